# baseline (speedup 1.0000x reference)
import functools

import jax
import jax.numpy as jnp
from jax import lax
from jax.experimental import pallas as pl
from jax.experimental.pallas import tpu as pltpu
from jax.experimental.pallas import tpu_sc as plsc

_NC, _NS, _L = 2, 16, 16
_NW = _NC * _NS
_CHX = 32
_CHG = 16


def _sc_gather_add(x_nc, idx_n, table):
    N, C = x_nc.shape
    n_per_w = N // _NW
    CHX, CHG = _CHX, _CHG
    HPC = CHX // CHG
    n_big = n_per_w // CHX
    mesh = plsc.VectorSubcoreMesh(core_axis_name="c", subcore_axis_name="s")

    scratch = [
        pltpu.VMEM((n_per_w,), jnp.int32),
        pltpu.VMEM((2, CHX, C), jnp.float32),
        pltpu.VMEM((HPC, CHG, C), jnp.float32),
        pltpu.SemaphoreType.DMA((2,)),
        pltpu.SemaphoreType.DMA((HPC,)),
        pltpu.SemaphoreType.DMA((2,)),
    ]

    @functools.partial(
        pl.kernel,
        out_type=jax.ShapeDtypeStruct((N, C), jnp.float32),
        mesh=mesh,
        scratch_types=scratch,
    )
    def k(x_hbm, idx_hbm, tab_hbm, out_hbm, idx_v, xb, rb, sx, sr, so):
        wid = lax.axis_index("s") * _NC + lax.axis_index("c")
        base = wid * n_per_w
        pltpu.sync_copy(idx_hbm.at[pl.ds(base, n_per_w)], idx_v)

        def issue_x(ci, b):
            off = base + ci * CHX
            pltpu.async_copy(x_hbm.at[pl.ds(off, CHX)], xb.at[b], sx.at[b])

        def wait_x(ci, b):
            off = base + ci * CHX
            pltpu.make_async_copy(x_hbm.at[pl.ds(off, CHX)], xb.at[b],
                                  sx.at[b]).wait()

        def issue_g(gi, h):
            pltpu.async_copy(tab_hbm.at[idx_v.at[pl.ds(gi * CHG, CHG)]],
                             rb.at[h], sr.at[h])

        def wait_g(gi, h):
            pltpu.make_async_copy(tab_hbm.at[idx_v.at[pl.ds(gi * CHG, CHG)]],
                                  rb.at[h], sr.at[h]).wait()

        def issue_out(ci, b):
            off = base + ci * CHX
            pltpu.async_copy(xb.at[b], out_hbm.at[pl.ds(off, CHX)], so.at[b])

        def wait_out(ci, b):
            off = base + ci * CHX
            pltpu.make_async_copy(xb.at[b], out_hbm.at[pl.ds(off, CHX)],
                                  so.at[b]).wait()

        def add_half(b, h):
            def row(i, c2):
                for j in range(C // _L):
                    sl = pl.ds(j * _L, _L)
                    plsc.addupdate(xb.at[b, h * CHG + i, sl], rb[h, i, sl])
                return c2

            lax.fori_loop(0, CHG, row, 0)

        issue_x(0, 0)
        for h in range(HPC):
            issue_g(h, h)

        def body(ci, carry):
            b = lax.rem(ci, 2)
            nb = lax.rem(ci + 1, 2)

            @pl.when(ci + 1 < n_big)
            def _():
                @pl.when(ci >= 1)
                def _():
                    wait_out(ci - 1, nb)

                issue_x(ci + 1, nb)

            wait_x(ci, b)
            for h in range(HPC):
                gi = HPC * ci + h
                wait_g(gi, h)
                add_half(b, h)
                ngi = HPC * (ci + 1) + h

                @pl.when(ngi < HPC * n_big)
                def _():
                    issue_g(ngi, h)

            issue_out(ci, b)
            return carry

        lax.fori_loop(0, n_big, body, 0)
        wait_out(n_big - 2, (n_big - 2) % 2)
        wait_out(n_big - 1, (n_big - 1) % 2)

    return k(x_nc, idx_n, table)


def kernel(x_btc, times_bt, embeddings_tc, offset):
    B, T, C = x_btc.shape
    x = x_btc.reshape(B * T, C)
    idx = times_bt.reshape(B * T).astype(jnp.int32)
    out = _sc_gather_add(x, idx, embeddings_tc)
    return out.reshape(B, T, C)


# speedup vs baseline: 1.1208x; 1.1208x over previous
import functools

import jax
import jax.numpy as jnp
from jax import lax
from jax.experimental import pallas as pl
from jax.experimental.pallas import tpu as pltpu
from jax.experimental.pallas import tpu_sc as plsc

_NC, _NS, _L = 2, 16, 16
_NW = _NC * _NS
_CHX = 32
_CHG = 16


def _sc_gather_add(x_nc, idx_n, table):
    N, C = x_nc.shape
    n_per_w = N // _NW
    CHX, CHG = _CHX, _CHG
    HPC = CHX // CHG
    n_big = n_per_w // CHX
    mesh = plsc.VectorSubcoreMesh(core_axis_name="c", subcore_axis_name="s")

    scratch = [
        pltpu.VMEM((n_per_w,), jnp.int32),
        pltpu.VMEM((CHX, C), jnp.float32),
        pltpu.VMEM((CHX, C), jnp.float32),
        pltpu.VMEM((HPC, CHG, C), jnp.float32),
        pltpu.SemaphoreType.DMA((2,)),
        pltpu.SemaphoreType.DMA((HPC,)),
        pltpu.SemaphoreType.DMA((2,)),
    ]

    @functools.partial(
        pl.kernel,
        out_type=jax.ShapeDtypeStruct((N, C), jnp.float32),
        mesh=mesh,
        scratch_types=scratch,
    )
    def k(x_hbm, idx_hbm, tab_hbm, out_hbm, idx_v, xb0, xb1, rb, sx, sr, so):
        wid = lax.axis_index("s") * _NC + lax.axis_index("c")
        base = wid * n_per_w
        pltpu.sync_copy(idx_hbm.at[pl.ds(base, n_per_w)], idx_v)

        def issue_x(ci, xr, b):
            off = base + ci * CHX
            pltpu.async_copy(x_hbm.at[pl.ds(off, CHX)], xr, sx.at[b])

        def wait_x(ci, xr, b):
            off = base + ci * CHX
            pltpu.make_async_copy(x_hbm.at[pl.ds(off, CHX)], xr,
                                  sx.at[b]).wait()

        def issue_g(gi, h):
            pltpu.async_copy(tab_hbm.at[idx_v.at[pl.ds(gi * CHG, CHG)]],
                             rb.at[h], sr.at[h])

        def wait_g(gi, h):
            pltpu.make_async_copy(tab_hbm.at[idx_v.at[pl.ds(gi * CHG, CHG)]],
                                  rb.at[h], sr.at[h]).wait()

        def issue_out(ci, xr, b):
            off = base + ci * CHX
            pltpu.async_copy(xr, out_hbm.at[pl.ds(off, CHX)], so.at[b])

        def wait_out(ci, xr, b):
            off = base + ci * CHX
            pltpu.make_async_copy(xr, out_hbm.at[pl.ds(off, CHX)],
                                  so.at[b]).wait()

        def add_half(xr, h):
            def row(i, c2):
                for j in range(C // _L):
                    sl = pl.ds(j * _L, _L)
                    plsc.addupdate(xr.at[h * CHG + i, sl], rb[h, i, sl])
                return c2

            lax.fori_loop(0, CHG, row, 0)

        issue_x(0, xb0, 0)
        for h in range(HPC):
            issue_g(h, h)

        def step(ci, cur, nxt, bc, bn):
            @pl.when(ci + 1 < n_big)
            def _():
                @pl.when(ci >= 1)
                def _():
                    wait_out(ci - 1, nxt, bn)

                issue_x(ci + 1, nxt, bn)

            wait_x(ci, cur, bc)
            for h in range(HPC):
                gi = HPC * ci + h
                wait_g(gi, h)
                add_half(cur, h)
                ngi = HPC * (ci + 1) + h

                @pl.when(ngi < HPC * n_big)
                def _():
                    issue_g(ngi, h)

            issue_out(ci, cur, bc)

        def body(g, carry):
            ci0 = 2 * g
            step(ci0, xb0, xb1, 0, 1)
            step(ci0 + 1, xb1, xb0, 1, 0)
            return carry

        lax.fori_loop(0, n_big // 2, body, 0)
        wait_out(n_big - 2, xb0, 0)
        wait_out(n_big - 1, xb1, 1)

    return k(x_nc, idx_n, table)


def kernel(x_btc, times_bt, embeddings_tc, offset):
    B, T, C = x_btc.shape
    x = x_btc.reshape(B * T, C)
    idx = times_bt.reshape(B * T).astype(jnp.int32)
    out = _sc_gather_add(x, idx, embeddings_tc)
    return out.reshape(B, T, C)


# speedup vs baseline: 1.8463x; 1.6473x over previous
import functools

import jax
import jax.numpy as jnp
from jax import lax
from jax.experimental import pallas as pl
from jax.experimental.pallas import tpu as pltpu
from jax.experimental.pallas import tpu_sc as plsc

_NC, _NS, _L = 2, 16, 16
_NW = _NC * _NS
_CH = 16
_NBUF = 3
_DEPTH = 2


def _sc_gather_add(x_nc, idx_n, table):
    N, C = x_nc.shape
    n_per_w = N // _NW
    CH, NB, D = _CH, _NBUF, _DEPTH
    n_ch = n_per_w // CH
    mesh = plsc.VectorSubcoreMesh(core_axis_name="c", subcore_axis_name="s")

    scratch = [
        pltpu.VMEM((n_per_w,), jnp.int32),
        pltpu.VMEM((NB, CH, C), jnp.float32),
        pltpu.VMEM((NB, CH, C), jnp.float32),
        pltpu.SemaphoreType.DMA((NB,)),
        pltpu.SemaphoreType.DMA((NB,)),
        pltpu.SemaphoreType.DMA((NB,)),
    ]

    @functools.partial(
        pl.kernel,
        out_type=jax.ShapeDtypeStruct((N, C), jnp.float32),
        mesh=mesh,
        scratch_types=scratch,
    )
    def k(x_hbm, idx_hbm, tab_hbm, out_hbm, idx_v, xb, rb, sx, sr, so):
        wid = lax.axis_index("s") * _NC + lax.axis_index("c")
        base = wid * n_per_w
        pltpu.sync_copy(idx_hbm.at[pl.ds(base, n_per_w)], idx_v)

        def issue_in(ci, b):
            off = base + ci * CH
            pltpu.async_copy(x_hbm.at[pl.ds(off, CH)], xb.at[b], sx.at[b])
            pltpu.async_copy(tab_hbm.at[idx_v.at[pl.ds(ci * CH, CH)]], rb.at[b],
                             sr.at[b])

        def wait_in(ci, b):
            off = base + ci * CH
            pltpu.make_async_copy(x_hbm.at[pl.ds(off, CH)], xb.at[b],
                                  sx.at[b]).wait()
            pltpu.make_async_copy(
                tab_hbm.at[idx_v.at[pl.ds(ci * CH, CH)]], rb.at[b],
                sr.at[b]).wait()

        def issue_out(ci, b):
            off = base + ci * CH
            pltpu.async_copy(xb.at[b], out_hbm.at[pl.ds(off, CH)], so.at[b])

        def wait_out(ci, b):
            off = base + ci * CH
            pltpu.make_async_copy(xb.at[b], out_hbm.at[pl.ds(off, CH)],
                                  so.at[b]).wait()

        def add_rows(b):
            def row(i, c2):
                for j in range(C // _L):
                    sl = pl.ds(j * _L, _L)
                    plsc.addupdate(xb.at[b, i, sl], rb[b, i, sl])
                return c2

            lax.fori_loop(0, CH, row, 0)

        for p in range(D):
            issue_in(p, p)

        def body(ci, carry):
            b = lax.rem(ci, NB)
            wait_in(ci, b)
            nci = ci + D
            nb = lax.rem(nci, NB)

            @pl.when(nci < n_ch)
            def _():
                @pl.when(nci >= NB)
                def _():
                    wait_out(nci - NB, nb)

                issue_in(nci, nb)

            add_rows(b)
            issue_out(ci, b)
            return carry

        lax.fori_loop(0, n_ch, body, 0)
        for t in range(NB):
            ci = n_ch - NB + t
            wait_out(ci, ci % NB)

    return k(x_nc, idx_n, table)


def kernel(x_btc, times_bt, embeddings_tc, offset):
    B, T, C = x_btc.shape
    x = x_btc.reshape(B * T, C)
    idx = times_bt.reshape(B * T).astype(jnp.int32)
    out = _sc_gather_add(x, idx, embeddings_tc)
    return out.reshape(B, T, C)


# speedup vs baseline: 1.8469x; 1.0003x over previous
import functools

import jax
import jax.numpy as jnp
from jax import lax
from jax.experimental import pallas as pl
from jax.experimental.pallas import tpu as pltpu
from jax.experimental.pallas import tpu_sc as plsc

_NC, _NS, _L = 2, 16, 16
_NW = _NC * _NS
_CH = 16
_NBUF = 3
_DEPTH = 2


def _sc_gather_add(x_nc, idx_n, table):
    N, C = x_nc.shape
    n_per_w = N // _NW
    CH, NB, D = _CH, _NBUF, _DEPTH
    n_ch = n_per_w // CH
    mesh = plsc.VectorSubcoreMesh(core_axis_name="c", subcore_axis_name="s")

    scratch = [
        pltpu.VMEM((n_per_w,), jnp.int32),
        pltpu.VMEM((NB, CH, C), jnp.float32),
        pltpu.VMEM((NB, CH, C), jnp.float32),
        pltpu.SemaphoreType.DMA((NB,)),
        pltpu.SemaphoreType.DMA((NB,)),
        pltpu.SemaphoreType.DMA((NB,)),
    ]

    @functools.partial(
        pl.kernel,
        out_type=jax.ShapeDtypeStruct((N, C), jnp.float32),
        mesh=mesh,
        scratch_types=scratch,
    )
    def k(x_hbm, idx_hbm, tab_hbm, out_hbm, idx_v, xb, rb, sx, sr, so):
        wid = lax.axis_index("s") * _NC + lax.axis_index("c")
        base = wid * n_per_w
        pltpu.sync_copy(idx_hbm.at[pl.ds(base, n_per_w)], idx_v)

        def issue_in(ci, b):
            off = (ci * _NW + wid) * CH
            pltpu.async_copy(x_hbm.at[pl.ds(off, CH)], xb.at[b], sx.at[b])
            pltpu.async_copy(tab_hbm.at[idx_v.at[pl.ds(ci * CH, CH)]], rb.at[b],
                             sr.at[b])

        def wait_in(ci, b):
            off = (ci * _NW + wid) * CH
            pltpu.make_async_copy(x_hbm.at[pl.ds(off, CH)], xb.at[b],
                                  sx.at[b]).wait()
            pltpu.make_async_copy(
                tab_hbm.at[idx_v.at[pl.ds(ci * CH, CH)]], rb.at[b],
                sr.at[b]).wait()

        def issue_out(ci, b):
            off = (ci * _NW + wid) * CH
            pltpu.async_copy(xb.at[b], out_hbm.at[pl.ds(off, CH)], so.at[b])

        def wait_out(ci, b):
            off = (ci * _NW + wid) * CH
            pltpu.make_async_copy(xb.at[b], out_hbm.at[pl.ds(off, CH)],
                                  so.at[b]).wait()

        def add_rows(b):
            def row(i, c2):
                for j in range(C // _L):
                    sl = pl.ds(j * _L, _L)
                    plsc.addupdate(xb.at[b, i, sl], rb[b, i, sl])
                return c2

            lax.fori_loop(0, CH, row, 0)

        for p in range(D):
            issue_in(p, p)

        def body(ci, carry):
            b = lax.rem(ci, NB)
            wait_in(ci, b)
            nci = ci + D
            nb = lax.rem(nci, NB)

            @pl.when(nci < n_ch)
            def _():
                @pl.when(nci >= NB)
                def _():
                    wait_out(nci - NB, nb)

                issue_in(nci, nb)

            add_rows(b)
            issue_out(ci, b)
            return carry

        lax.fori_loop(0, n_ch, body, 0)
        for t in range(NB):
            ci = n_ch - NB + t
            wait_out(ci, ci % NB)

    return k(x_nc, idx_n, table)


def kernel(x_btc, times_bt, embeddings_tc, offset):
    B, T, C = x_btc.shape
    x = x_btc.reshape(B * T, C)
    idx = times_bt.reshape(B * T).astype(jnp.int32)
    n_step = (B * T) // (_NW * _CH)
    idx = idx.reshape(n_step, _NW, _CH).transpose(1, 0, 2).reshape(B * T)
    out = _sc_gather_add(x, idx, embeddings_tc)
    return out.reshape(B, T, C)
